# speedup vs baseline: 4.1785x; 4.1785x over previous
import functools

import jax
import jax.numpy as jnp
from jax import lax
from jax.experimental import pallas as pl
from jax.experimental.pallas import tpu as pltpu
from jax.experimental.pallas import tpu_sc as plsc

_B, _S, _G = 4, 2048, 4
_VOCAB, _DIM = 100000, 128
_NC, _NS = 2, 16
_NW = _NC * _NS
_ROWS = _B * _S
_RPW = _ROWS // _NW
_GPW = _RPW * _G
_CHUNK = 128
_NCHUNK = _GPW // _CHUNK
_OPC = _CHUNK // _G

_mesh = plsc.VectorSubcoreMesh(core_axis_name="c", subcore_axis_name="s")


@functools.partial(
    pl.kernel,
    mesh=_mesh,
    out_type=jax.ShapeDtypeStruct((_ROWS, _DIM), jnp.float32),
    scratch_types=[
        pltpu.VMEM((_GPW,), jnp.int32),
        pltpu.VMEM((_CHUNK, _DIM), jnp.float32),
        pltpu.VMEM((_CHUNK, _DIM), jnp.float32),
        pltpu.VMEM((_RPW, _DIM), jnp.float32),
        pltpu.SemaphoreType.DMA,
        pltpu.SemaphoreType.DMA,
    ],
)
def _embed_sum(x_hbm, tab_hbm, out_hbm, idx_v, rows_a, rows_b, out_v, sem_a, sem_b):
    wid = lax.axis_index("s") * _NC + lax.axis_index("c")
    pltpu.sync_copy(x_hbm.at[pl.ds(wid * _GPW, _GPW)], idx_v)

    off = (lax.iota(jnp.int32, 16) % _G) * _VOCAB
    for i in range(_GPW // 16):
        sl = pl.ds(i * 16, 16)
        idx_v[sl] = idx_v[sl] + off

    bufs = (rows_a, rows_b)
    sems = (sem_a, sem_b)

    def start(j):
        return pltpu.async_copy(
            tab_hbm.at[idx_v.at[pl.ds(j * _CHUNK, _CHUNK)]],
            bufs[j % 2],
            sems[j % 2],
        )

    cp = start(0)
    for j in range(_NCHUNK):
        nxt = start(j + 1) if j + 1 < _NCHUNK else None
        cp.wait()
        buf = bufs[j % 2]

        def row_body(r, carry, j=j, buf=buf):
            for c in range(_DIM // 16):
                sl = pl.ds(c * 16, 16)
                v = (buf[4 * r, sl] + buf[4 * r + 1, sl]) + (
                    buf[4 * r + 2, sl] + buf[4 * r + 3, sl]
                )
                out_v[j * _OPC + r, sl] = v
            return carry

        lax.fori_loop(0, _OPC, row_body, 0)
        cp = nxt

    pltpu.sync_copy(out_v, out_hbm.at[pl.ds(wid * _RPW, _RPW)])


def kernel(x, tables):
    xf = x.reshape(_ROWS * _G)
    tf = tables.reshape(_G * _VOCAB, _DIM)
    out = _embed_sum(xf, tf)
    return out.reshape(_B, _S, _DIM)
